# baseline (speedup 1.0000x reference)
import jax
import jax.numpy as jnp
from jax.experimental import pallas as pl

_EPS = 1e-5
_N = 32768
_B = 16


def _fused_kernel(x_ref, starts_ref, lens_ref, cw_ref, cb_ref, g1_ref, b1_ref,
                  fcw_ref, fcb_ref, g2_ref, b2_ref, out_ref):
    x = x_ref[...]
    cw = cw_ref[...]

    out1 = jnp.sum(x * cw, axis=1, keepdims=True) + cb_ref[0, 0]
    mu1 = jnp.mean(out1)
    var1 = jnp.mean((out1 - mu1) ** 2)
    out1 = (out1 - mu1) / jnp.sqrt(var1 + _EPS) * g1_ref[0, 0] + b1_ref[0, 0]

    s = jnp.mean(x * out1, axis=1, keepdims=True)

    row = jax.lax.broadcasted_iota(jnp.int32, (_N, _B), 0)
    starts = starts_ref[...]
    lens = lens_ref[...]
    mask = (row >= starts) & (row < starts + lens)

    sb = jnp.where(mask, s, -jnp.inf)
    m = jnp.max(sb, axis=0, keepdims=True)
    e = jnp.where(mask, jnp.exp(sb - m), 0.0)
    denom = jnp.sum(e, axis=0, keepdims=True)

    w = e * (1.0 / (denom * lens.astype(jnp.float32)))
    pooled = jax.lax.dot_general(
        w, x, dimension_numbers=(((0,), (0,)), ((), ())),
        preferred_element_type=jnp.float32)

    res = jax.lax.dot_general(
        pooled, fcw_ref[...], dimension_numbers=(((1,), (1,)), ((), ())),
        preferred_element_type=jnp.float32) + fcb_ref[...]

    mu2 = jnp.mean(res, axis=0, keepdims=True)
    var2 = jnp.mean((res - mu2) ** 2, axis=0, keepdims=True)
    out_ref[...] = (res - mu2) / jnp.sqrt(var2 + _EPS) * g2_ref[...] + b2_ref[...]


def kernel(x, length, conv_w, conv_b, bn1_gamma, bn1_beta, fc_w, fc_b,
           bn2_gamma, bn2_beta):
    starts = jnp.concatenate(
        [jnp.zeros((1,), dtype=length.dtype), jnp.cumsum(length)[:-1]])
    return pl.pallas_call(
        _fused_kernel,
        out_shape=jax.ShapeDtypeStruct((_B, 64), jnp.float32),
    )(
        x,
        starts.reshape(1, _B),
        length.reshape(1, _B),
        conv_w.reshape(1, 32),
        conv_b.reshape(1, 1),
        bn1_gamma.reshape(1, 1),
        bn1_beta.reshape(1, 1),
        fc_w,
        fc_b.reshape(1, 64),
        bn2_gamma.reshape(1, 64),
        bn2_beta.reshape(1, 64),
    )


# speedup vs baseline: 1.8515x; 1.8515x over previous
import jax
import jax.numpy as jnp
from jax.experimental import pallas as pl

_EPS = 1e-5
_N = 32768
_B = 16


def _fused_kernel(x_ref, starts_ref, lens_ref, w2_ref, cb_ref, g1_ref, b1_ref,
                  fcwt_ref, fcb_ref, g2_ref, b2_ref, out_ref):
    x = x_ref[...]

    sp = jax.lax.dot_general(
        w2_ref[...], x, dimension_numbers=(((1,), (1,)), ((), ())),
        preferred_element_type=jnp.float32)
    out1 = sp[0:1, :] + cb_ref[0, 0]
    rmean = sp[1:2, :]

    mu1 = jnp.mean(out1)
    d = out1 - mu1
    var1 = jnp.mean(d * d)
    out1n = d / jnp.sqrt(var1 + _EPS) * g1_ref[0, 0] + b1_ref[0, 0]

    s = out1n * rmean

    m = jnp.max(s)
    e = jnp.exp(s - m)

    lane = jax.lax.broadcasted_iota(jnp.int32, (_B, _N), 1)
    starts = starts_ref[...]
    lens = lens_ref[...]
    mask = (lane >= starts) & (lane < starts + lens)
    me = jnp.where(mask, e, 0.0)

    denom = jnp.sum(me, axis=1, keepdims=True)
    pooled = jax.lax.dot_general(
        me, x, dimension_numbers=(((1,), (0,)), ((), ())),
        preferred_element_type=jnp.float32)
    pooled = pooled * (1.0 / (denom * lens.astype(jnp.float32)))

    res = jax.lax.dot_general(
        pooled, fcwt_ref[...], dimension_numbers=(((1,), (0,)), ((), ())),
        preferred_element_type=jnp.float32) + fcb_ref[...]

    mu2 = jnp.mean(res, axis=0, keepdims=True)
    var2 = jnp.mean((res - mu2) ** 2, axis=0, keepdims=True)
    out_ref[...] = (res - mu2) / jnp.sqrt(var2 + _EPS) * g2_ref[...] + b2_ref[...]


def kernel(x, length, conv_w, conv_b, bn1_gamma, bn1_beta, fc_w, fc_b,
           bn2_gamma, bn2_beta):
    starts = jnp.concatenate(
        [jnp.zeros((1,), dtype=length.dtype), jnp.cumsum(length)[:-1]])
    w2 = jnp.zeros((8, 32), jnp.float32)
    w2 = w2.at[0, :].set(conv_w[0]).at[1, :].set(1.0 / 32.0)
    return pl.pallas_call(
        _fused_kernel,
        out_shape=jax.ShapeDtypeStruct((_B, 64), jnp.float32),
    )(
        x,
        starts.reshape(_B, 1),
        length.reshape(_B, 1),
        w2,
        conv_b.reshape(1, 1),
        bn1_gamma.reshape(1, 1),
        bn1_beta.reshape(1, 1),
        fc_w.T,
        fc_b.reshape(1, 64),
        bn2_gamma.reshape(1, 64),
        bn2_beta.reshape(1, 64),
    )
